# baseline (speedup 1.0000x reference)
import math

import jax
import jax.numpy as jnp
import numpy as np
from jax.experimental import pallas as pl

_B, _S, _D = 2, 8192, 768
_E, _K, _H = 64, 8, 12
_HD = _D // _H
_TSA = 4096
_TSB = 512


def _row_sum64(a):
    n = a.shape[0]
    acc = jax.lax.slice(a, (0, 0), (n, 8))
    for t in range(1, 8):
        acc = acc + jax.lax.slice(a, (0, 8 * t), (n, 8 * t + 8))
    acc = jax.lax.slice(acc, (0, 0), (n, 4)) + jax.lax.slice(acc, (0, 4), (n, 8))
    acc = jax.lax.slice(acc, (0, 0), (n, 2)) + jax.lax.slice(acc, (0, 2), (n, 4))
    acc = jax.lax.slice(acc, (0, 0), (n, 1)) + jax.lax.slice(acc, (0, 1), (n, 2))
    return acc


def _q_block(x_ref, wq_ref, bq_ref, q_ref):
    q = jnp.dot(x_ref[...], wq_ref[...], preferred_element_type=jnp.float32)
    q_ref[...] = (q + bq_ref[...]).astype(jnp.bfloat16)


def _attn_block(q_ref, kt_ref, v_ref, ctx_ref):
    f32 = jnp.float32
    bf16 = jnp.bfloat16
    ts = q_ref.shape[0]
    qb = q_ref[...]
    parts = []
    for h in range(_H):
        qh = jax.lax.slice(qb, (0, _HD * h), (ts, _HD * (h + 1)))
        sh = jnp.dot(qh, kt_ref[h], preferred_element_type=f32)
        sh = sh * f32(0.125)
        mh = jnp.max(sh, axis=1, keepdims=True)
        ph = jnp.exp(sh - mh)
        ah = ph / _row_sum64(ph)
        parts.append(
            jnp.dot(ah.astype(bf16), v_ref[h], preferred_element_type=f32))
    ctx_ref[...] = jnp.concatenate(parts, axis=1).astype(bf16)


def _gate_block(ctx_ref, outw_ref, outb_ref, g1w_ref, g1b_ref, g2w_ref,
                g2b_ref, idx_ref, w_ref):
    f32 = jnp.float32
    bf16 = jnp.bfloat16

    attn_out = jnp.dot(ctx_ref[...], outw_ref[...], preferred_element_type=f32)
    attn_out = attn_out + outb_ref[...]

    h1 = jnp.dot(attn_out.astype(bf16), g1w_ref[...],
                 preferred_element_type=f32)
    h1 = h1 + g1b_ref[...]
    h1 = h1 * (jax.lax.erf(h1 / np.float32(np.sqrt(2.0))) + 1.0) * f32(0.5)

    logits = jnp.dot(h1.astype(bf16), g2w_ref[...], preferred_element_type=f32)
    logits = logits + g2b_ref[...]

    lm = jnp.max(logits, axis=1, keepdims=True)
    el = jnp.exp(logits - lm)
    probs = el / _row_sum64(el)

    lane = jax.lax.broadcasted_iota(jnp.int32, probs.shape, 1)
    vals = probs
    top_v = []
    top_i = []
    for _ in range(_K):
        mv = jnp.max(vals, axis=1, keepdims=True)
        cand = jnp.where(vals == mv, lane, _E)
        mi = jnp.min(cand, axis=1, keepdims=True)
        top_v.append(mv)
        top_i.append(mi)
        vals = jnp.where(lane == mi, -1.0, vals)
    tv = jnp.concatenate(top_v, axis=1)
    ti = jnp.concatenate(top_i, axis=1)
    tw = tv / (jnp.sum(tv, axis=1, keepdims=True) + 1e-8)

    idx_ref[...] = ti
    w_ref[...] = tw


def kernel(x, expert_embeddings, in_proj_weight, in_proj_bias,
           out_proj_weight, out_proj_bias, gate_w1, gate_b1, gate_w2,
           gate_b2):
    f32 = jnp.float32
    bf16 = jnp.bfloat16
    d = _D
    wq = in_proj_weight[:d]
    wk = in_proj_weight[d:2 * d]
    wv = in_proj_weight[2 * d:]
    bq = in_proj_bias[:d]
    bk = in_proj_bias[d:2 * d]
    bv = in_proj_bias[2 * d:]

    k = expert_embeddings @ wk.T + bk
    v = expert_embeddings @ wv.T + bv
    kt3 = (k.reshape(_E, _H, _HD).transpose(1, 2, 0)).astype(bf16)
    v3 = (v.reshape(_E, _H, _HD).transpose(1, 0, 2)).astype(bf16)

    n = _B * _S
    x2 = x.reshape(n, d).astype(bf16)

    tok = lambda i: (i, 0)
    whole = lambda i: (0, 0)
    whole3 = lambda i: (0, 0, 0)

    qb = pl.pallas_call(
        _q_block,
        grid=(n // _TSA,),
        in_specs=[pl.BlockSpec((_TSA, d), tok),
                  pl.BlockSpec((d, d), whole),
                  pl.BlockSpec((1, d), whole)],
        out_specs=pl.BlockSpec((_TSA, d), tok),
        out_shape=jax.ShapeDtypeStruct((n, d), bf16),
    )(x2, wq.T.astype(bf16), bq.reshape(1, d))

    ctx = pl.pallas_call(
        _attn_block,
        grid=(n // _TSB,),
        in_specs=[pl.BlockSpec((_TSB, d), tok),
                  pl.BlockSpec((_H, _HD, _E), whole3),
                  pl.BlockSpec((_H, _E, _HD), whole3)],
        out_specs=pl.BlockSpec((_TSB, d), tok),
        out_shape=jax.ShapeDtypeStruct((n, d), bf16),
    )(qb, kt3, v3)

    idx2, w2 = pl.pallas_call(
        _gate_block,
        grid=(n // _TSA,),
        in_specs=[pl.BlockSpec((_TSA, d), tok),
                  pl.BlockSpec((d, d), whole),
                  pl.BlockSpec((1, d), whole),
                  pl.BlockSpec((d, _E * 4), whole),
                  pl.BlockSpec((1, _E * 4), whole),
                  pl.BlockSpec((_E * 4, _E), whole),
                  pl.BlockSpec((1, _E), whole)],
        out_specs=(pl.BlockSpec((_TSA, _K), tok),
                   pl.BlockSpec((_TSA, _K), tok)),
        out_shape=(jax.ShapeDtypeStruct((n, _K), jnp.int32),
                   jax.ShapeDtypeStruct((n, _K), f32)),
    )(ctx, out_proj_weight.T.astype(bf16), out_proj_bias.reshape(1, d),
      gate_w1.T.astype(bf16), gate_b1.reshape(1, _E * 4),
      gate_w2.T.astype(bf16), gate_b2.reshape(1, _E))

    return (idx2.reshape(_B, _S, _K), w2.reshape(_B, _S, _K))
